# Initial kernel scaffold; baseline (speedup 1.0000x reference)
import jax
import jax.numpy as jnp
from jax.experimental import pallas as pl


def kernel(inputs, proxies, labels, classes, proxy_centers, class_centers, label2proxy, cam2proxy):
    raise NotImplementedError("write your pallas kernel here")


# speedup vs baseline: 2.5285x; 2.5285x over previous
import functools
import jax
import jax.numpy as jnp
from jax.experimental import pallas as pl
from jax.experimental.pallas import tpu as pltpu

_B, _D = 1024, 128
_NP, _NC = 20000, 10000
_TEMP = 0.07
_K = 58
_PMAX = 8
_RB = 128
_NEG = -1e30


def _body(x_ref, pcT_ref, ccT_ref, sp_ref, cls_ref, c2p_ref, out_ref, mnp_ref):
    f32 = jnp.float32
    x = x_ref[...]
    scores = jax.lax.dot(x, pcT_ref[...], precision=jax.lax.Precision.HIGHEST,
                         preferred_element_type=f32) * (1.0 / _TEMP)
    valid = jnp.sum(c2p_ref[...], axis=0, keepdims=True) > 0.0
    col = jax.lax.broadcasted_iota(jnp.int32, (_RB, _NP), 1)
    sp = sp_ref[...]
    posmask = col == sp[:, 0:1]
    for q in range(1, _PMAX):
        posmask = posmask | (col == sp[:, q:q + 1])
    vp = posmask & valid
    n_vp = jnp.sum(vp.astype(f32), axis=1, keepdims=True)
    pos_sum_s = jnp.sum(jnp.where(vp, scores, 0.0), axis=1, keepdims=True)
    pos_max = jnp.max(jnp.where(vp, scores, _NEG), axis=1, keepdims=True)
    mnp0 = jnp.where(valid & (~posmask), scores, _NEG)
    mnp_ref[...] = mnp0
    m0 = jnp.max(mnp0, axis=1, keepdims=True)
    big_m = jnp.maximum(jnp.maximum(m0, pos_max), -1e4)
    pos_exp = jnp.sum(jnp.where(vp, jnp.exp(scores - big_m), 0.0),
                      axis=1, keepdims=True)

    budget0 = jnp.float32(_K) - n_vp

    def step(_, carry):
        acc, budget = carry
        cur = mnp_ref[...]
        m = jnp.max(cur, axis=1, keepdims=True)
        eq = cur == m
        cnt = jnp.sum(eq.astype(f32), axis=1, keepdims=True)
        take = jnp.minimum(cnt, budget)
        acc = acc + take * jnp.exp(jnp.maximum(m - big_m, _NEG))
        budget = budget - take
        mnp_ref[...] = jnp.where(eq, _NEG, cur)
        return acc, budget

    zeros = jnp.zeros((_RB, 1), f32)
    acc, _ = jax.lax.fori_loop(0, _K, step, (zeros, budget0))

    denom = jnp.maximum(pos_exp + acc, 1e-30)
    lse = big_m + jnp.log(denom)
    per_proxy = jnp.where(
        n_vp > 0.0,
        -(pos_sum_s - n_vp * lse) / jnp.maximum(n_vp, 1.0),
        0.0)

    cs = jax.lax.dot(x, ccT_ref[...], precision=jax.lax.Precision.HIGHEST,
                     preferred_element_type=f32) * (1.0 / _TEMP)
    cmax = jnp.max(cs, axis=1, keepdims=True)
    csum = jnp.sum(jnp.exp(cs - cmax), axis=1, keepdims=True)
    clse = cmax + jnp.log(csum)
    ccol = jax.lax.broadcasted_iota(jnp.int32, (_RB, _NC), 1)
    cid = cls_ref[...]
    own = jnp.sum(jnp.where(ccol == cid, cs, 0.0), axis=1, keepdims=True)
    per_class = clse - own
    out_ref[...] = per_proxy + per_class


@functools.partial(jax.jit, static_argnames=())
def kernel(inputs, proxies, labels, classes, proxy_centers, class_centers,
           label2proxy, cam2proxy):
    del proxies
    s_proxies = label2proxy[labels].astype(jnp.int32)
    cls2d = classes.astype(jnp.int32).reshape(_B, 1)
    pcT = proxy_centers.T
    ccT = class_centers.T
    nblk = _B // _RB
    per_row = pl.pallas_call(
        _body,
        grid=(nblk,),
        in_specs=[
            pl.BlockSpec((_RB, _D), lambda i: (i, 0)),
            pl.BlockSpec((_D, _NP), lambda i: (0, 0)),
            pl.BlockSpec((_D, _NC), lambda i: (0, 0)),
            pl.BlockSpec((_RB, _PMAX), lambda i: (i, 0)),
            pl.BlockSpec((_RB, 1), lambda i: (i, 0)),
            pl.BlockSpec((8, _NP), lambda i: (0, 0)),
        ],
        out_specs=pl.BlockSpec((_RB, 1), lambda i: (i, 0)),
        out_shape=jax.ShapeDtypeStruct((_B, 1), jnp.float32),
        scratch_shapes=[pltpu.VMEM((_RB, _NP), jnp.float32)],
    )(inputs, pcT, ccT, s_proxies, cls2d, cam2proxy)
    return jnp.mean(per_row)


# speedup vs baseline: 2.6719x; 1.0567x over previous
import functools
import jax
import jax.numpy as jnp
from jax.experimental import pallas as pl
from jax.experimental.pallas import tpu as pltpu

_B, _D = 1024, 128
_NP, _NC = 20000, 10000
_TEMP = 0.07
_K = 58
_PMAX = 8
_RB = 128
_NEG = -1e30


def _body(x_ref, pcT_ref, ccT_ref, sp_ref, cls_ref, c2p_ref, out_ref, mnp_ref):
    f32 = jnp.float32
    x = x_ref[...]
    scores = jax.lax.dot(x, pcT_ref[...], precision=jax.lax.Precision.HIGHEST,
                         preferred_element_type=f32) * (1.0 / _TEMP)
    valid = jnp.sum(c2p_ref[...], axis=0, keepdims=True) > 0.0
    col = jax.lax.broadcasted_iota(jnp.int32, (_RB, _NP), 1)
    sp = sp_ref[...]
    posmask = col == sp[:, 0:1]
    for q in range(1, _PMAX):
        posmask = posmask | (col == sp[:, q:q + 1])
    vp = posmask & valid
    n_vp = jnp.sum(vp.astype(f32), axis=1, keepdims=True)
    pos_sum_s = jnp.sum(jnp.where(vp, scores, 0.0), axis=1, keepdims=True)
    pos_max = jnp.max(jnp.where(vp, scores, _NEG), axis=1, keepdims=True)
    mnp0 = jnp.where(valid & (~posmask), scores, _NEG)
    mnp_ref[...] = mnp0
    m0 = jnp.max(mnp0, axis=1, keepdims=True)
    big_m = jnp.maximum(jnp.maximum(m0, pos_max), -1e4)
    pos_exp = jnp.sum(jnp.where(vp, jnp.exp(scores - big_m), 0.0),
                      axis=1, keepdims=True)

    budget0 = jnp.float32(_K) - n_vp

    def step(_, carry):
        acc, budget, m_prev = carry
        cur = mnp_ref[...]
        eq = cur == m_prev
        cnt = jnp.sum(eq.astype(f32), axis=1, keepdims=True)
        nxt = jnp.where(eq, _NEG, cur)
        m = jnp.max(nxt, axis=1, keepdims=True)
        mnp_ref[...] = nxt
        take = jnp.minimum(cnt, budget)
        acc = acc + take * jnp.exp(jnp.maximum(m_prev - big_m, _NEG))
        budget = budget - take
        return acc, budget, m

    zeros = jnp.zeros((_RB, 1), f32)
    acc, _, _ = jax.lax.fori_loop(0, _K, step, (zeros, budget0, m0))

    denom = jnp.maximum(pos_exp + acc, 1e-30)
    lse = big_m + jnp.log(denom)
    per_proxy = jnp.where(
        n_vp > 0.0,
        -(pos_sum_s - n_vp * lse) / jnp.maximum(n_vp, 1.0),
        0.0)

    cs = jax.lax.dot(x, ccT_ref[...], precision=jax.lax.Precision.HIGHEST,
                     preferred_element_type=f32) * (1.0 / _TEMP)
    cmax = jnp.max(cs, axis=1, keepdims=True)
    csum = jnp.sum(jnp.exp(cs - cmax), axis=1, keepdims=True)
    clse = cmax + jnp.log(csum)
    ccol = jax.lax.broadcasted_iota(jnp.int32, (_RB, _NC), 1)
    cid = cls_ref[...]
    own = jnp.sum(jnp.where(ccol == cid, cs, 0.0), axis=1, keepdims=True)
    per_class = clse - own
    out_ref[...] = per_proxy + per_class


@functools.partial(jax.jit, static_argnames=())
def kernel(inputs, proxies, labels, classes, proxy_centers, class_centers,
           label2proxy, cam2proxy):
    del proxies
    s_proxies = label2proxy[labels].astype(jnp.int32)
    cls2d = classes.astype(jnp.int32).reshape(_B, 1)
    pcT = proxy_centers.T
    ccT = class_centers.T
    nblk = _B // _RB
    per_row = pl.pallas_call(
        _body,
        grid=(nblk,),
        in_specs=[
            pl.BlockSpec((_RB, _D), lambda i: (i, 0)),
            pl.BlockSpec((_D, _NP), lambda i: (0, 0)),
            pl.BlockSpec((_D, _NC), lambda i: (0, 0)),
            pl.BlockSpec((_RB, _PMAX), lambda i: (i, 0)),
            pl.BlockSpec((_RB, 1), lambda i: (i, 0)),
            pl.BlockSpec((8, _NP), lambda i: (0, 0)),
        ],
        out_specs=pl.BlockSpec((_RB, 1), lambda i: (i, 0)),
        out_shape=jax.ShapeDtypeStruct((_B, 1), jnp.float32),
        scratch_shapes=[pltpu.VMEM((_RB, _NP), jnp.float32)],
    )(inputs, pcT, ccT, s_proxies, cls2d, cam2proxy)
    return jnp.mean(per_row)


# speedup vs baseline: 8.5833x; 3.2125x over previous
import jax
import jax.numpy as jnp
from jax.experimental import pallas as pl
from jax.experimental.pallas import tpu as pltpu

_B, _D = 1024, 128
_NP, _NC = 20000, 10000
_TEMP = 0.07
_K = 58
_PMAX = 8
_RB = 128
_CH = 2500
_NEG = -1e30
_BISECT_ITERS = 10


def _body(xT_ref, pc_ref, cc_ref, spT_ref, clsT_ref, c2pT_ref, out_ref, w_ref):
    f32 = jnp.float32
    inv_t = 1.0 / _TEMP
    xT = xT_ref[...]
    sp = spT_ref[...]

    n_vp = jnp.zeros((1, _RB), f32)
    pos_sum_s = jnp.zeros((1, _RB), f32)
    pos_acc = jnp.zeros((1, _RB), f32)
    pm = jnp.full((1, _RB), _NEG, f32)
    m0 = jnp.full((1, _RB), _NEG, f32)
    mmin = jnp.full((1, _RB), 1e30, f32)

    for c in range(_NP // _CH):
        sTc = jax.lax.dot(pc_ref[pl.ds(c * _CH, _CH), :], xT,
                          precision=jax.lax.Precision.DEFAULT,
                          preferred_element_type=f32) * inv_t
        validc = jnp.sum(c2pT_ref[pl.ds(c * _CH, _CH), :].astype(jnp.float32),
                         axis=1, keepdims=True) > 0.0
        rowc = jax.lax.broadcasted_iota(jnp.int32, (_CH, _RB), 0) + c * _CH
        posm = rowc == sp[0:1, :]
        for q in range(1, _PMAX):
            posm = posm | (rowc == sp[q:q + 1, :])
        vpc = posm & validc
        n_vp = n_vp + jnp.sum(vpc.astype(f32), axis=0, keepdims=True)
        pos_sum_s = pos_sum_s + jnp.sum(jnp.where(vpc, sTc, 0.0),
                                        axis=0, keepdims=True)
        pmc = jnp.max(jnp.where(vpc, sTc, _NEG), axis=0, keepdims=True)
        npm = jnp.maximum(pm, pmc)
        pos_acc = (pos_acc * jnp.exp(jnp.minimum(pm - npm, 0.0))
                   + jnp.sum(jnp.where(vpc, jnp.exp(jnp.minimum(sTc - npm, 0.0)),
                                       0.0), axis=0, keepdims=True))
        pm = npm
        wc = jnp.where(validc & (~posm), sTc, _NEG)
        w_ref[pl.ds(c * _CH, _CH), :] = wc
        m0 = jnp.maximum(m0, jnp.max(wc, axis=0, keepdims=True))
        mmin = jnp.minimum(mmin, jnp.min(jnp.where(wc == _NEG, 1e30, wc),
                                         axis=0, keepdims=True))

    big_m = jnp.maximum(jnp.maximum(m0, pm), -1e4)
    pos_exp = pos_acc * jnp.exp(jnp.minimum(pm - big_m, 0.0))
    budget = jnp.float32(_K) - n_vp

    lo0 = jnp.minimum(mmin, m0)
    hi0 = m0 + 1.0

    def step(_, carry):
        lo, hi = carry
        d = (hi - lo) * 0.25
        t1, t2, t3 = lo + d, lo + 2.0 * d, lo + 3.0 * d
        c1 = jnp.zeros((1, _RB), f32)
        c2 = jnp.zeros((1, _RB), f32)
        c3 = jnp.zeros((1, _RB), f32)
        for c in range(_NP // _CH):
            wv = w_ref[pl.ds(c * _CH, _CH), :]
            c1 = c1 + jnp.sum((wv >= t1).astype(f32), axis=0, keepdims=True)
            c2 = c2 + jnp.sum((wv >= t2).astype(f32), axis=0, keepdims=True)
            c3 = c3 + jnp.sum((wv >= t3).astype(f32), axis=0, keepdims=True)
        ge1, ge2, ge3 = c1 >= budget, c2 >= budget, c3 >= budget
        nlo = jnp.where(ge3, t3, jnp.where(ge2, t2, jnp.where(ge1, t1, lo)))
        nhi = jnp.where(~ge1, t1, jnp.where(~ge2, t2, jnp.where(~ge3, t3, hi)))
        return nlo, nhi

    lo, hi = jax.lax.fori_loop(0, _BISECT_ITERS, step, (lo0, hi0))

    c_hi = jnp.zeros((1, _RB), f32)
    s_hi = jnp.zeros((1, _RB), f32)
    for c in range(_NP // _CH):
        wv = w_ref[pl.ds(c * _CH, _CH), :]
        gehi = wv >= hi
        c_hi = c_hi + jnp.sum(gehi.astype(f32), axis=0, keepdims=True)
        s_hi = s_hi + jnp.sum(
            jnp.where(gehi, jnp.exp(jnp.minimum(wv - big_m, 0.0)), 0.0),
            axis=0, keepdims=True)
    s_band = (budget - c_hi) * jnp.exp(
        jnp.minimum((lo + hi) * 0.5 - big_m, 0.0))
    denom = jnp.maximum(pos_exp + s_hi + s_band, 1e-30)
    lse = big_m + jnp.log(denom)
    per_proxy = jnp.where(
        n_vp > 0.0,
        -(pos_sum_s - n_vp * lse) / jnp.maximum(n_vp, 1.0),
        0.0)

    cid = clsT_ref[...].reshape(1, _RB)
    cm = jnp.full((1, _RB), _NEG, f32)
    cacc = jnp.zeros((1, _RB), f32)
    own = jnp.zeros((1, _RB), f32)
    for c in range(_NC // _CH):
        csc = jax.lax.dot(cc_ref[pl.ds(c * _CH, _CH), :], xT,
                          precision=jax.lax.Precision.DEFAULT,
                          preferred_element_type=f32) * inv_t
        chm = jnp.max(csc, axis=0, keepdims=True)
        ncm = jnp.maximum(cm, chm)
        cacc = (cacc * jnp.exp(jnp.minimum(cm - ncm, 0.0))
                + jnp.sum(jnp.exp(jnp.minimum(csc - ncm, 0.0)),
                          axis=0, keepdims=True))
        cm = ncm
        crow = jax.lax.broadcasted_iota(jnp.int32, (_CH, _RB), 0) + c * _CH
        own = own + jnp.sum(jnp.where(crow == cid, csc, 0.0),
                            axis=0, keepdims=True)
    clse = cm + jnp.log(cacc)
    out_ref[...] = (per_proxy + clse - own).reshape(1, 1, _RB)


def kernel(inputs, proxies, labels, classes, proxy_centers, class_centers,
           label2proxy, cam2proxy):
    del proxies
    spT = label2proxy[labels].astype(jnp.int32).T
    cls3d = classes.astype(jnp.int32).reshape(_B // _RB, 1, _RB)
    xT = inputs.T
    c2pT = (cam2proxy.T > 0.0).astype(jnp.int8)
    nblk = _B // _RB
    per_row = pl.pallas_call(
        _body,
        grid=(nblk,),
        in_specs=[
            pl.BlockSpec((_D, _RB), lambda i: (0, i)),
            pl.BlockSpec((_NP, _D), lambda i: (0, 0)),
            pl.BlockSpec((_NC, _D), lambda i: (0, 0)),
            pl.BlockSpec((_PMAX, _RB), lambda i: (0, i)),
            pl.BlockSpec((1, 1, _RB), lambda i: (i, 0, 0)),
            pl.BlockSpec((_NP, 8), lambda i: (0, 0)),
        ],
        out_specs=pl.BlockSpec((1, 1, _RB), lambda i: (i, 0, 0)),
        out_shape=jax.ShapeDtypeStruct((nblk, 1, _RB), jnp.float32),
        scratch_shapes=[pltpu.VMEM((_NP, _RB), jnp.float32)],
    )(xT, proxy_centers, class_centers, spT, cls3d, c2pT)
    return jnp.mean(per_row)
